# baseline (speedup 1.0000x reference)
import functools

import jax
import jax.numpy as jnp
from jax import lax
from jax.experimental import pallas as pl
from jax.experimental.pallas import tpu as pltpu
from jax.experimental.pallas import tpu_sc as plsc

NC = 2
NS = 16
LANES = 16
NW = NC * NS
K = 64
DEPTH = 4


_GATHER_DNUMS = lax.GatherDimensionNumbers(
    offset_dims=(), collapsed_slice_dims=(0,), start_index_map=(0,))


def _bcast_lane(vec, l):
    idx = jnp.full((LANES, 1), l, dtype=jnp.int32)
    return lax.gather(vec, idx, _GATHER_DNUMS, (1,),
                      mode=lax.GatherScatterMode.PROMISE_IN_BOUNDS)


def _mm1_body(x_ref, w_ref, b_ref, o_ref):
    o_ref[...] = lax.dot_general(
        x_ref[...], w_ref[...], (((1,), (1,)), ((), ())),
        preferred_element_type=jnp.float32,
    ) + b_ref[...]


def _mm2_body(p0_ref, p1_ref, w_ref, b_ref, o_ref):
    h = jnp.maximum(p0_ref[...] + p1_ref[...], 0.0)
    o_ref[...] = lax.dot_general(
        h, w_ref[...], (((1,), (1,)), ((), ())),
        preferred_element_type=jnp.float32,
    ) + b_ref[...]


def _add2_body(q0_ref, q1_ref, o_ref):
    o_ref[...] = q0_ref[...] + q1_ref[...]


def _mm1(x, w, b, bn):
    n, d = x.shape
    return pl.pallas_call(
        _mm1_body,
        grid=(n // bn,),
        in_specs=[
            pl.BlockSpec((bn, d), lambda i: (i, 0)),
            pl.BlockSpec(w.shape, lambda i: (0, 0)),
            pl.BlockSpec((1, d), lambda i: (0, 0)),
        ],
        out_specs=pl.BlockSpec((bn, w.shape[0]), lambda i: (i, 0)),
        out_shape=jax.ShapeDtypeStruct((n, w.shape[0]), jnp.float32),
    )(x, w, b.reshape(1, -1))


def _mm2(p0, p1, w, b, bn):
    n, d = p0.shape
    return pl.pallas_call(
        _mm2_body,
        grid=(n // bn,),
        in_specs=[
            pl.BlockSpec((bn, d), lambda i: (i, 0)),
            pl.BlockSpec((bn, d), lambda i: (i, 0)),
            pl.BlockSpec(w.shape, lambda i: (0, 0)),
            pl.BlockSpec((1, d), lambda i: (0, 0)),
        ],
        out_specs=pl.BlockSpec((bn, w.shape[0]), lambda i: (i, 0)),
        out_shape=jax.ShapeDtypeStruct((n, w.shape[0]), jnp.float32),
    )(p0, p1, w, b.reshape(1, -1))


def _add2(q0, q1, bn):
    n, d = q0.shape
    return pl.pallas_call(
        _add2_body,
        grid=(n // bn,),
        in_specs=[
            pl.BlockSpec((bn, d), lambda i: (i, 0)),
            pl.BlockSpec((bn, d), lambda i: (i, 0)),
        ],
        out_specs=pl.BlockSpec((bn, d), lambda i: (i, 0)),
        out_shape=jax.ShapeDtypeStruct((n, d), jnp.float32),
    )(q0, q1)


def _make_spmm(n_acc, d, n_chunks):
    assert n_chunks % DEPTH == 0 and n_chunks >= 2 * DEPTH
    rpt = n_acc // NS
    mesh = plsc.VectorSubcoreMesh(core_axis_name="c", subcore_axis_name="s")

    @functools.partial(
        pl.kernel,
        out_type=jax.ShapeDtypeStruct((NC, n_acc, d), jnp.float32),
        mesh=mesh,
        scratch_types=[
            pltpu.VMEM((DEPTH, 2, K), jnp.int32),
            pltpu.VMEM((DEPTH, K), jnp.float32),
            pltpu.VMEM((DEPTH, K, d), jnp.float32),
            pltpu.VMEM_SHARED((n_acc, d), jnp.float32),
            pltpu.SemaphoreType.DMA((DEPTH,)),
            pltpu.SemaphoreType.DMA((DEPTH,)),
            pltpu.SemaphoreType.DMA((DEPTH,)),
        ],
    )
    def spmm(h_hbm, epk_hbm, vals_hbm, zero_hbm, out_hbm, ebuf, vbuf, rows,
             acc, isem, gsem, ssem):
        c = lax.axis_index("c")
        s = lax.axis_index("s")
        wid = s * NC + c
        pltpu.sync_copy(zero_hbm.at[pl.ds(s * rpt, rpt)],
                        acc.at[pl.ds(s * rpt, rpt)])
        plsc.subcore_barrier()

        def issue_idx(ci, q):
            pltpu.async_copy(epk_hbm.at[wid, ci], ebuf.at[q], isem.at[q])
            pltpu.async_copy(vals_hbm.at[wid, ci], vbuf.at[q], isem.at[q])

        def wait_idx(ci, q):
            pltpu.make_async_copy(epk_hbm.at[wid, ci], ebuf.at[q],
                                  isem.at[q]).wait()
            pltpu.make_async_copy(vals_hbm.at[wid, ci], vbuf.at[q],
                                  isem.at[q]).wait()

        def issue_gather(q):
            pltpu.async_copy(h_hbm.at[ebuf.at[q, 0]], rows.at[q], gsem.at[q])

        def wait_gather(q):
            pltpu.make_async_copy(h_hbm.at[ebuf.at[q, 0]], rows.at[q],
                                  gsem.at[q]).wait()

        def issue_scatter(q):
            pltpu.async_copy(rows.at[q], acc.at[ebuf.at[q, 1]], ssem.at[q],
                             add=True)

        def wait_scatter(q):
            pltpu.make_async_copy(rows.at[q], acc.at[ebuf.at[q, 1]],
                                  ssem.at[q]).wait()

        issue_idx(0, 0)
        issue_idx(1, 1)
        wait_idx(0, 0)
        issue_gather(0)

        def quarter(i, q):
            ci = i * DEPTH + q
            qn = (q + 1) % DEPTH
            qn2 = (q + 2) % DEPTH
            wait_gather(q)

            @pl.when(ci >= 2)
            def _():
                wait_scatter(qn2)

            @pl.when(ci + 2 < n_chunks)
            def _():
                issue_idx(ci + 2, qn2)

            def grp_body(g, gcarry):
                vv = vbuf[q, pl.ds(g * LANES, LANES)]
                for l in range(LANES):
                    val = _bcast_lane(vv, l)
                    e = g * LANES + l
                    for j in range(d // LANES):
                        sl = (q, e, pl.ds(j * LANES, LANES))
                        rows[sl] = rows[sl] * val
                return gcarry

            lax.fori_loop(0, K // LANES, grp_body, 0)

            @pl.when(ci + 1 < n_chunks)
            def _():
                wait_idx(ci + 1, qn)
                issue_gather(qn)

            issue_scatter(q)

        def iter_body(i, carry):
            for q in range(DEPTH):
                quarter(i, q)
            return carry

        lax.fori_loop(0, n_chunks // DEPTH, iter_body, 0)
        wait_scatter(DEPTH - 2)
        wait_scatter(DEPTH - 1)
        plsc.subcore_barrier()
        pltpu.sync_copy(acc.at[pl.ds(s * rpt, rpt)],
                        out_hbm.at[c, pl.ds(s * rpt, rpt)])

    return spmm


def kernel(x, edge_index, edge_values, W1, b1, W2, b2):
    n, d_in = x.shape
    e = edge_values.shape[0]
    cpd = NW * K * DEPTH
    n_chunks = DEPTH * (-(-e // cpd))
    e_pad = NW * K * n_chunks
    pad = e_pad - e

    dst = edge_index[0].astype(jnp.int32)
    src = edge_index[1].astype(jnp.int32)
    vals = edge_values
    if pad:
        dst = jnp.pad(dst, (0, pad))
        src = jnp.pad(src, (0, pad))
        vals = jnp.pad(vals, (0, pad))
    epk = jnp.stack([src, dst]).reshape(2, NW, n_chunks, K)
    epk = jnp.transpose(epk, (1, 2, 0, 3))
    valsp = vals.reshape(NW, n_chunks, K)

    n_acc = NS * (-(-n // (NS * 8)) * 8)
    zeros = jnp.zeros((n_acc, d_in), jnp.float32)
    spmm = _make_spmm(n_acc, d_in, n_chunks)

    h1 = _mm1(x, W1, b1, 1000)
    p = spmm(h1, epk, valsp, zeros)
    h2 = _mm2(p[0, :n], p[1, :n], W2, b2, 1000)
    q = spmm(h2, epk, valsp, zeros)
    return _add2(q[0, :n], q[1, :n], 1000)
